# baseline (speedup 1.0000x reference)
import functools
import operator

import numpy as np
import jax
import jax.numpy as jnp
from jax import lax
from jax.experimental import pallas as pl
from jax.experimental.pallas import tpu as pltpu
from jax.experimental.pallas import tpu_sc as plsc

_EPS = 0.1
_ROWS = 64
_COLS = 1_000_000
_NW = 32
_CHUNK = 31_264
_LANES = 16
_TINY = np.float32(np.finfo(np.float32).tiny)
_LN2 = np.float32(0.6931471805599453)
_SQRT2 = np.float32(1.4142135623730951)
_INT_MAX = np.int32(2**31 - 1)


def _np_threefry2x32(k0, k1, x0, x1):
    u32 = np.uint32
    x0 = np.asarray(x0, dtype=u32).copy()
    x1 = np.asarray(x1, dtype=u32).copy()
    ks = [u32(k0), u32(k1), u32(u32(k0) ^ u32(k1) ^ u32(0x1BD11BDA))]
    rotations = [[13, 15, 26, 6], [17, 29, 16, 24]]
    x0 = (x0 + ks[0]).astype(u32)
    x1 = (x1 + ks[1]).astype(u32)
    for i in range(5):
        for r in rotations[i % 2]:
            x0 = (x0 + x1).astype(u32)
            x1 = ((x1 << u32(r)) | (x1 >> u32(32 - r))).astype(u32)
            x1 = (x1 ^ x0).astype(u32)
        x0 = (x0 + ks[(i + 1) % 3]).astype(u32)
        x1 = (x1 + ks[(i + 2) % 3] + u32(i + 1)).astype(u32)
    return x0, x1


def _derive_constants():
    kd1 = _np_threefry2x32(0, 42, [0], [0])
    kd2 = _np_threefry2x32(0, 42, [0], [1])
    kd1 = (int(kd1[0][0]), int(kd1[1][0]))
    kd2 = (int(kd2[0][0]), int(kd2[1][0]))
    o0, o1 = _np_threefry2x32(kd2[0], kd2[1],
                              np.zeros(_ROWS, np.uint32),
                              np.arange(_ROWS, dtype=np.uint32))
    bits = (o0 ^ o1).astype(np.uint32)
    u = (((bits >> np.uint32(9)) | np.uint32(0x3F800000))
         .view(np.float32) - np.float32(1.0))
    sampled = np.where(u < np.float32(_EPS))[0]
    return kd1, tuple(int(r) for r in sampled)


_KD1, _SAMPLED_ROWS = _derive_constants()


def _rotl(x, r):
    return (x << np.uint32(r)) | (x >> np.uint32(32 - r))


def _threefry_bits(n_u32):
    k0, k1 = _KD1
    ks0 = np.uint32(k0)
    ks1 = np.uint32(k1)
    ks2 = np.uint32(ks0 ^ ks1 ^ np.uint32(0x1BD11BDA))
    ks = [ks0, ks1, ks2]
    rotations = [[13, 15, 26, 6], [17, 29, 16, 24]]
    x0 = jnp.full((_LANES,), ks0, jnp.uint32)
    x1 = n_u32 + ks1
    for i in range(5):
        for r in rotations[i % 2]:
            x0 = x0 + x1
            x1 = _rotl(x1, r)
            x1 = x1 ^ x0
        x0 = x0 + ks[(i + 1) % 3]
        x1 = x1 + np.uint32(int(ks[(i + 2) % 3]) + i + 1 & 0xFFFFFFFF)
    return x0 ^ x1


def _log_f32(u):
    bits = lax.bitcast_convert_type(u, jnp.uint32)
    e = (bits >> np.uint32(23)).astype(jnp.int32) - 127
    m = lax.bitcast_convert_type(
        (bits & np.uint32(0x007FFFFF)) | np.uint32(0x3F800000), jnp.float32)
    big = m >= _SQRT2
    m = jnp.where(big, m * np.float32(0.5), m)
    e = e + jnp.where(big, 1, 0)
    s = (m - np.float32(1.0)) / (m + np.float32(1.0))
    z = s * s
    p = np.float32(2.0 / 7.0) + z * np.float32(2.0 / 9.0)
    p = np.float32(2.0 / 5.0) + z * p
    p = np.float32(2.0 / 3.0) + z * p
    p = np.float32(2.0) + z * p
    return e.astype(jnp.float32) * _LN2 + s * p


def _lane_argmax(vb, ib):
    mx = jnp.max(vb)
    cand = jnp.where(vb == mx, ib, _INT_MAX)
    return mx, jnp.min(cand)


def _sc_body(input_hbm, pv_hbm, pi_hbm,
             buf0, buf1, stage_v, stage_i, sem0, sem1):
    cid = lax.axis_index("c")
    sid = lax.axis_index("s")
    wid = sid * 2 + cid
    col0 = jnp.minimum(wid * _CHUNK, _COLS - _CHUNK)
    iota = lax.broadcasted_iota(jnp.int32, (_LANES,), 0)

    def copy(r, buf, sem):
        return pltpu.make_async_copy(
            input_hbm.at[r, pl.ds(col0, _CHUNK)], buf, sem)

    def greedy_scan(buf):
        def body(i, carry):
            vb, ib, g = carry
            for u_ in range(2):
                v = buf[pl.ds((2 * i + u_) * _LANES, _LANES)]
                m = v > vb
                vb = jnp.where(m, v, vb)
                ib = jnp.where(m, g, ib)
                g = g + _LANES
            return vb, ib, g
        vb = jnp.full((_LANES,), -1.0, jnp.float32)
        ib = jnp.zeros((_LANES,), jnp.int32)
        vb, ib, _ = lax.fori_loop(0, _CHUNK // (2 * _LANES), body,
                                  (vb, ib, col0 + iota))
        return _lane_argmax(vb, ib)

    def sampled_scan(buf, r):
        nbase = r * _COLS + col0
        def body(i, carry):
            vb, ib, g = carry
            v = buf[pl.ds(i * _LANES, _LANES)]
            bits = _threefry_bits((nbase + g).astype(jnp.uint32))
            u = lax.bitcast_convert_type(
                (bits >> np.uint32(9)) | np.uint32(0x3F800000),
                jnp.float32) - np.float32(1.0)
            u = jnp.maximum(u, _TINY)
            w = v / (-_log_f32(u))
            gc = col0 + g
            m = w > vb
            vb = jnp.where(m, w, vb)
            ib = jnp.where(m, gc, ib)
            return vb, ib, g + _LANES
        vb = jnp.full((_LANES,), -1.0, jnp.float32)
        ib = jnp.zeros((_LANES,), jnp.int32)
        vb, ib, _ = lax.fori_loop(0, _CHUNK // _LANES, body, (vb, ib, iota))
        return _lane_argmax(vb, ib)

    def process(r, lane, buf, av, ai):
        is_s = functools.reduce(
            operator.or_, [r == rr for rr in _SAMPLED_ROWS])
        val, idx = lax.cond(is_s,
                            lambda: sampled_scan(buf, r),
                            lambda: greedy_scan(buf))
        m = iota == lane
        av = jnp.where(m, jnp.full((_LANES,), val, jnp.float32), av)
        ai = jnp.where(m, jnp.full((_LANES,), idx, jnp.int32), ai)
        return av, ai

    copy(0, buf0, sem0).start()
    for grp in range(4):
        def pair_body(j, carry, grp=grp):
            av, ai = carry
            r0 = grp * 16 + 2 * j
            copy(r0 + 1, buf1, sem1).start()
            copy(r0, buf0, sem0).wait()
            av, ai = process(r0, 2 * j, buf0, av, ai)
            copy((r0 + 2) & 63, buf0, sem0).start()
            copy(r0 + 1, buf1, sem1).wait()
            av, ai = process(r0 + 1, 2 * j + 1, buf1, av, ai)
            return av, ai
        av = jnp.zeros((_LANES,), jnp.float32)
        ai = jnp.zeros((_LANES,), jnp.int32)
        av, ai = lax.fori_loop(0, 8, pair_body, (av, ai))
        stage_v[pl.ds(grp * 16, _LANES)] = av
        stage_i[pl.ds(grp * 16, _LANES)] = ai
    copy(0, buf0, sem0).wait()
    pltpu.sync_copy(stage_v, pv_hbm.at[wid])
    pltpu.sync_copy(stage_i, pi_hbm.at[wid])


_sc_call = pl.kernel(
    _sc_body,
    out_type=(jax.ShapeDtypeStruct((_NW, _ROWS), jnp.float32),
              jax.ShapeDtypeStruct((_NW, _ROWS), jnp.int32)),
    mesh=plsc.VectorSubcoreMesh(core_axis_name="c", subcore_axis_name="s",
                                num_cores=2, num_subcores=16),
    scratch_types=[
        pltpu.VMEM((_CHUNK,), jnp.float32),
        pltpu.VMEM((_CHUNK,), jnp.float32),
        pltpu.VMEM((_ROWS,), jnp.float32),
        pltpu.VMEM((_ROWS,), jnp.int32),
        pltpu.SemaphoreType.DMA,
        pltpu.SemaphoreType.DMA,
    ],
    compiler_params=pltpu.CompilerParams(use_tc_tiling_on_sc=False,
                                         needs_layout_passes=False),
)


def _merge_body(pv_ref, pi_ref, out_ref):
    v = pv_ref[...]
    i = pi_ref[...]
    mx = jnp.max(v, axis=0, keepdims=True)
    cand = jnp.where(v == mx, i, _INT_MAX)
    out_ref[...] = jnp.min(cand, axis=0, keepdims=True)


_merge_call = pl.pallas_call(
    _merge_body,
    out_shape=jax.ShapeDtypeStruct((1, _ROWS), jnp.int32),
)


def kernel(input):
    pv, pi = _sc_call(input)
    return _merge_call(pv, pi).reshape(_ROWS)
